# baseline (speedup 1.0000x reference)
import functools

import jax
import jax.numpy as jnp
from jax import lax
from jax.experimental import pallas as pl
from jax.experimental.pallas import tpu as pltpu
from jax.experimental.pallas import tpu_sc as plsc

NN = 10000
NP = 10240
NE = 320000
D = 128
EW = 128
NWIN = NE // EW


def _deg_body(src_hbm, dst_hbm, degp_hbm, idx_v, ones_v, stage_v,
              degs_sh, degd_sh):
    c = lax.axis_index("c")
    s = lax.axis_index("s")
    wid = c * 16 + s
    for j in range(8):
        ones_v[pl.ds(j * 16, 16)] = jnp.ones((16,), jnp.float32)
    zed = NP // 16
    def zrow(r, _):
        stage_v[pl.ds(r * 16, 16)] = jnp.zeros((16,), jnp.float32)
        return 0
    lax.fori_loop(0, zed // 16, zrow, 0)
    pltpu.sync_copy(stage_v, degs_sh.at[pl.ds(s * zed, zed)])
    pltpu.sync_copy(stage_v, degd_sh.at[pl.ds(s * zed, zed)])
    plsc.subcore_barrier()

    nbase, nrem = NWIN // 32, NWIN % 32
    nw = nbase + (wid < nrem).astype(jnp.int32)

    def body(i, _):
        base = (wid + i * 32) * EW
        pltpu.sync_copy(src_hbm.at[pl.ds(base, EW)], idx_v)
        pltpu.sync_copy(ones_v, degs_sh.at[idx_v], add=True)
        pltpu.sync_copy(dst_hbm.at[pl.ds(base, EW)], idx_v)
        pltpu.sync_copy(ones_v, degd_sh.at[idx_v], add=True)
        return 0
    lax.fori_loop(0, nw, body, 0)
    plsc.subcore_barrier()

    pltpu.sync_copy(degs_sh.at[pl.ds(s * zed, zed)], stage_v)
    pltpu.sync_copy(stage_v, degp_hbm.at[c, 0, pl.ds(s * zed, zed)])
    pltpu.sync_copy(degd_sh.at[pl.ds(s * zed, zed)], stage_v)
    pltpu.sync_copy(stage_v, degp_hbm.at[c, 1, pl.ds(s * zed, zed)])


def _deg_call(src, dst):
    mesh = plsc.VectorSubcoreMesh(core_axis_name="c", subcore_axis_name="s")
    zed = NP // 16
    f = functools.partial(
        pl.kernel,
        out_type=jax.ShapeDtypeStruct((2, 2, NP), jnp.float32),
        mesh=mesh,
        compiler_params=pltpu.CompilerParams(needs_layout_passes=False),
        scratch_types=[
            pltpu.VMEM((EW,), jnp.int32),
            pltpu.VMEM((EW,), jnp.float32),
            pltpu.VMEM((zed,), jnp.float32),
            pltpu.VMEM_SHARED((NP,), jnp.float32),
            pltpu.VMEM_SHARED((NP,), jnp.float32),
        ],
    )(_deg_body)
    return f(src, dst)


def _enc_body(x_ref, we_ref, ea_ref, wedge_ref, degp_ref,
              y_ref, ew_ref, inv_ref):
    y_ref[...] = jnp.dot(x_ref[...], we_ref[...],
                         preferred_element_type=jnp.float32)
    ea = ea_ref[...]
    z = (ea[0] * wedge_ref[0, 0] + ea[1] * wedge_ref[1, 0]
         + ea[2] * wedge_ref[2, 0] + ea[3] * wedge_ref[3, 0])
    ew_ref[...] = jax.nn.sigmoid(z)
    deg = degp_ref[...]
    degsum = deg[0] + deg[1]
    inv = jnp.where(degsum > 0,
                    lax.rsqrt(jnp.maximum(degsum, 1e-12)),
                    jnp.zeros_like(degsum))
    inv_ref[...] = inv


def _enc_call(x, W_enc, ea_T, W_edge, degp):
    return pl.pallas_call(
        _enc_body,
        out_shape=[
            jax.ShapeDtypeStruct((NN, D), jnp.float32),
            jax.ShapeDtypeStruct((NWIN, EW), jnp.float32),
            jax.ShapeDtypeStruct((2, NP), jnp.float32),
        ],
        in_specs=[
            pl.BlockSpec(memory_space=pltpu.VMEM),
            pl.BlockSpec(memory_space=pltpu.VMEM),
            pl.BlockSpec(memory_space=pltpu.VMEM),
            pl.BlockSpec(memory_space=pltpu.SMEM),
            pl.BlockSpec(memory_space=pltpu.VMEM),
        ],
    )(x, W_enc, ea_T, W_edge, degp)


def _agg_body(y_hbm, src_hbm, dst_hbm, ew_hbm, invs_hbm, invd_hbm, out_hbm,
              sidx, didx, ewv, ginvs, ginvd, rows, agg_sh,
              semi, semg, sems, seminv):
    c = lax.axis_index("c")
    s = lax.axis_index("s")
    wid = c * 16 + s

    def zrow(r, _):
        for j in range(8):
            rows[0, r, pl.ds(j * 16, 16)] = jnp.zeros((16,), jnp.float32)
        return 0
    lax.fori_loop(0, EW, zrow, 0)
    for t in range(NP // 16 // EW):
        pltpu.sync_copy(rows.at[0],
                        agg_sh.at[pl.ds((s * 5 + t) * EW, EW)])
    plsc.subcore_barrier()

    def issue_idx(slot, w):
        base = w * EW
        pltpu.async_copy(src_hbm.at[pl.ds(base, EW)], sidx.at[slot],
                         semi.at[slot])
        pltpu.async_copy(dst_hbm.at[pl.ds(base, EW)], didx.at[slot],
                         semi.at[slot])
        pltpu.async_copy(ew_hbm.at[pl.ds(base, EW)], ewv.at[slot],
                         semi.at[slot])

    def wait_idx(slot):
        pltpu.make_async_copy(src_hbm.at[pl.ds(0, EW)], sidx.at[slot],
                              semi.at[slot]).wait()
        pltpu.make_async_copy(dst_hbm.at[pl.ds(0, EW)], didx.at[slot],
                              semi.at[slot]).wait()
        pltpu.make_async_copy(ew_hbm.at[pl.ds(0, EW)], ewv.at[slot],
                              semi.at[slot]).wait()

    def start_gather(slot, half):
        pltpu.async_copy(y_hbm.at[sidx.at[slot]], rows.at[half], semg.at[half])
        pltpu.async_copy(invs_hbm.at[sidx.at[slot]], ginvs.at[slot],
                         seminv.at[slot])
        pltpu.async_copy(invd_hbm.at[didx.at[slot]], ginvd.at[slot],
                         seminv.at[slot])

    def wait_gather(slot, half):
        pltpu.make_async_copy(y_hbm.at[pl.ds(0, EW)],
                              rows.at[half],
                              semg.at[half]).wait()
        pltpu.make_async_copy(invs_hbm.at[pl.ds(0, EW)], ginvs.at[slot],
                              seminv.at[slot]).wait()
        pltpu.make_async_copy(invd_hbm.at[pl.ds(0, EW)], ginvd.at[slot],
                              seminv.at[slot]).wait()

    def start_scatter(slot, half):
        return pltpu.async_copy(rows.at[half],
                                agg_sh.at[didx.at[slot]], sems.at[half],
                                add=True)

    def wait_scatter(slot, half):
        pltpu.make_async_copy(rows.at[half],
                              agg_sh.at[pl.ds(0, EW)], sems.at[half]).wait()

    def scale(slot, half):
        def chunk(cc, _):
            sl = pl.ds(cc * 16, 16)
            co = ginvs[slot, sl] * ginvd[slot, sl] * ewv[slot, sl]
            for l in range(16):
                r = cc * 16 + l
                cb = jnp.full((16,), co[l], jnp.float32)
                for j in range(8):
                    sl2 = pl.ds(j * 16, 16)
                    rows[half, r, sl2] = rows[half, r, sl2] * cb
            return 0
        lax.fori_loop(0, 8, chunk, 0)

    nbase, nrem = NWIN // 32, NWIN % 32
    nw = nbase + (wid < nrem).astype(jnp.int32)

    issue_idx(0, wid)

    def body(i, _):
        par = lax.rem(i, 2)
        slot = lax.rem(i, 3)

        @pl.when(i >= 2)
        def _():
            wait_scatter(lax.rem(i + 1, 3), par)

        @pl.when(i + 1 < nw)
        def _():
            issue_idx(lax.rem(i + 1, 3), wid + (i + 1) * 32)

        @pl.when(i < nw)
        def _():
            wait_idx(slot)
            start_gather(slot, par)

        @pl.when(i >= 1)
        def _():
            pslot = lax.rem(i + 2, 3)
            ppar = 1 - par
            wait_gather(pslot, ppar)
            scale(pslot, ppar)
            start_scatter(pslot, ppar)
        return 0
    lax.fori_loop(0, nw + 1, body, 0)
    wait_scatter(lax.rem(nw + 2, 3), lax.rem(nw + 1, 2))

    plsc.subcore_barrier()

    for t in range(NP // 16 // EW):
        off = (s * 5 + t) * EW
        pltpu.sync_copy(agg_sh.at[pl.ds(off, EW)], rows.at[0])
        pltpu.sync_copy(rows.at[0], out_hbm.at[c, pl.ds(off, EW)])


def _agg_call(y, src, dst, ewf, inv_s, inv_d):
    mesh = plsc.VectorSubcoreMesh(core_axis_name="c", subcore_axis_name="s")
    f = functools.partial(
        pl.kernel,
        out_type=jax.ShapeDtypeStruct((2, NP, D), jnp.float32),
        mesh=mesh,
        compiler_params=pltpu.CompilerParams(needs_layout_passes=False),
        scratch_types=[
            pltpu.VMEM((3, EW), jnp.int32),
            pltpu.VMEM((3, EW), jnp.int32),
            pltpu.VMEM((3, EW), jnp.float32),
            pltpu.VMEM((3, EW), jnp.float32),
            pltpu.VMEM((3, EW), jnp.float32),
            pltpu.VMEM((2, EW, D), jnp.float32),
            pltpu.VMEM_SHARED((NP, D), jnp.float32),
            pltpu.SemaphoreType.DMA((3,)),
            pltpu.SemaphoreType.DMA((2,)),
            pltpu.SemaphoreType.DMA((2,)),
            pltpu.SemaphoreType.DMA((3,)),
        ],
    )(_agg_body)
    return f(y, src, dst, ewf, inv_s, inv_d)


def _dec_body(aggp_ref, benc_ref, wd_ref, bdec_ref, out_ref):
    z = aggp_ref[0] + aggp_ref[1] + benc_ref[...]
    h = z * jax.nn.sigmoid(z)
    out_ref[...] = (jnp.dot(h, wd_ref[...], preferred_element_type=jnp.float32)
                    + bdec_ref[...])


def _dec_call(aggp, b_enc, W_dec, b_dec):
    return pl.pallas_call(
        _dec_body,
        out_shape=jax.ShapeDtypeStruct((NP, D), jnp.float32),
    )(aggp, b_enc, W_dec, b_dec)


def kernel(x, edge_index, edge_attr, W_edge, W_enc, b_enc, W_dec, b_dec):
    src = edge_index[0].astype(jnp.int32)
    dst = edge_index[1].astype(jnp.int32)
    ea_T = edge_attr.T.reshape(4, NWIN, EW)

    degp = _deg_call(src, dst)
    y, ew, inv2 = _enc_call(x, W_enc, ea_T, W_edge, degp)
    aggp = _agg_call(y, src, dst, ew.reshape(-1), inv2[0], inv2[1])
    out = _dec_call(aggp, b_enc, W_dec, b_dec)
    return out[:NN]


# speedup vs baseline: 1.0052x; 1.0052x over previous
import functools

import jax
import jax.numpy as jnp
from jax import lax
from jax.experimental import pallas as pl
from jax.experimental.pallas import tpu as pltpu
from jax.experimental.pallas import tpu_sc as plsc

NN = 10000
NP = 10240
NE = 320000
D = 128
EW = 128
NWIN = NE // EW


def _deg_body(src_hbm, dst_hbm, degp_hbm, idx_v, ones_v, stage_v,
              degs_sh, degd_sh):
    c = lax.axis_index("c")
    s = lax.axis_index("s")
    wid = c * 16 + s
    for j in range(8):
        ones_v[pl.ds(j * 16, 16)] = jnp.ones((16,), jnp.float32)
    zed = NP // 16
    def zrow(r, _):
        stage_v[pl.ds(r * 16, 16)] = jnp.zeros((16,), jnp.float32)
        return 0
    lax.fori_loop(0, zed // 16, zrow, 0)
    pltpu.sync_copy(stage_v, degs_sh.at[pl.ds(s * zed, zed)])
    pltpu.sync_copy(stage_v, degd_sh.at[pl.ds(s * zed, zed)])
    plsc.subcore_barrier()

    nbase, nrem = NWIN // 32, NWIN % 32
    nw = nbase + (wid < nrem).astype(jnp.int32)

    def body(i, _):
        base = (wid + i * 32) * EW
        pltpu.sync_copy(src_hbm.at[pl.ds(base, EW)], idx_v)
        pltpu.sync_copy(ones_v, degs_sh.at[idx_v], add=True)
        pltpu.sync_copy(dst_hbm.at[pl.ds(base, EW)], idx_v)
        pltpu.sync_copy(ones_v, degd_sh.at[idx_v], add=True)
        return 0
    lax.fori_loop(0, nw, body, 0)
    plsc.subcore_barrier()

    pltpu.sync_copy(degs_sh.at[pl.ds(s * zed, zed)], stage_v)
    pltpu.sync_copy(stage_v, degp_hbm.at[c, 0, pl.ds(s * zed, zed)])
    pltpu.sync_copy(degd_sh.at[pl.ds(s * zed, zed)], stage_v)
    pltpu.sync_copy(stage_v, degp_hbm.at[c, 1, pl.ds(s * zed, zed)])


def _deg_call(src, dst):
    mesh = plsc.VectorSubcoreMesh(core_axis_name="c", subcore_axis_name="s")
    zed = NP // 16
    f = functools.partial(
        pl.kernel,
        out_type=jax.ShapeDtypeStruct((2, 2, NP), jnp.float32),
        mesh=mesh,
        compiler_params=pltpu.CompilerParams(needs_layout_passes=False),
        scratch_types=[
            pltpu.VMEM((EW,), jnp.int32),
            pltpu.VMEM((EW,), jnp.float32),
            pltpu.VMEM((zed,), jnp.float32),
            pltpu.VMEM_SHARED((NP,), jnp.float32),
            pltpu.VMEM_SHARED((NP,), jnp.float32),
        ],
    )(_deg_body)
    return f(src, dst)


def _enc_body(x_ref, we_ref, ea_ref, wedge_ref, degp_ref,
              y_ref, ew_ref, inv_ref):
    deg = degp_ref[...]
    degsum = deg[0] + deg[1]
    inv = jnp.where(degsum > 0,
                    lax.rsqrt(jnp.maximum(degsum, 1e-12)),
                    jnp.zeros_like(degsum))
    inv_ref[...] = inv
    xs = x_ref[...] * inv[0, :NN, None]
    y_ref[...] = jnp.dot(xs, we_ref[...],
                         preferred_element_type=jnp.float32)
    ea = ea_ref[...]
    z = (ea[0] * wedge_ref[0, 0] + ea[1] * wedge_ref[1, 0]
         + ea[2] * wedge_ref[2, 0] + ea[3] * wedge_ref[3, 0])
    ew_ref[...] = jax.nn.sigmoid(z)


def _enc_call(x, W_enc, ea_T, W_edge, degp):
    return pl.pallas_call(
        _enc_body,
        out_shape=[
            jax.ShapeDtypeStruct((NN, D), jnp.float32),
            jax.ShapeDtypeStruct((NWIN, EW), jnp.float32),
            jax.ShapeDtypeStruct((2, NP), jnp.float32),
        ],
        in_specs=[
            pl.BlockSpec(memory_space=pltpu.VMEM),
            pl.BlockSpec(memory_space=pltpu.VMEM),
            pl.BlockSpec(memory_space=pltpu.VMEM),
            pl.BlockSpec(memory_space=pltpu.SMEM),
            pl.BlockSpec(memory_space=pltpu.VMEM),
        ],
    )(x, W_enc, ea_T, W_edge, degp)


def _agg_body(y_hbm, src_hbm, dst_hbm, ew_hbm, out_hbm,
              sidx, didx, ewv, rows, agg_sh,
              semi, semg, sems):
    c = lax.axis_index("c")
    s = lax.axis_index("s")
    wid = c * 16 + s

    def zrow(r, _):
        for j in range(8):
            rows[0, r, pl.ds(j * 16, 16)] = jnp.zeros((16,), jnp.float32)
        return 0
    lax.fori_loop(0, EW, zrow, 0)
    for t in range(NP // 16 // EW):
        pltpu.sync_copy(rows.at[0],
                        agg_sh.at[pl.ds((s * 5 + t) * EW, EW)])
    plsc.subcore_barrier()

    def issue_idx(slot, w):
        base = w * EW
        pltpu.async_copy(src_hbm.at[pl.ds(base, EW)], sidx.at[slot],
                         semi.at[slot])
        pltpu.async_copy(dst_hbm.at[pl.ds(base, EW)], didx.at[slot],
                         semi.at[slot])
        pltpu.async_copy(ew_hbm.at[pl.ds(base, EW)], ewv.at[slot],
                         semi.at[slot])

    def wait_idx(slot):
        pltpu.make_async_copy(src_hbm.at[pl.ds(0, EW)], sidx.at[slot],
                              semi.at[slot]).wait()
        pltpu.make_async_copy(dst_hbm.at[pl.ds(0, EW)], didx.at[slot],
                              semi.at[slot]).wait()
        pltpu.make_async_copy(ew_hbm.at[pl.ds(0, EW)], ewv.at[slot],
                              semi.at[slot]).wait()

    def start_gather(slot, half):
        pltpu.async_copy(y_hbm.at[sidx.at[slot]], rows.at[half], semg.at[half])

    def wait_gather(slot, half):
        pltpu.make_async_copy(y_hbm.at[pl.ds(0, EW)],
                              rows.at[half],
                              semg.at[half]).wait()

    def start_scatter(slot, half):
        return pltpu.async_copy(rows.at[half],
                                agg_sh.at[didx.at[slot]], sems.at[half],
                                add=True)

    def wait_scatter(slot, half):
        pltpu.make_async_copy(rows.at[half],
                              agg_sh.at[pl.ds(0, EW)], sems.at[half]).wait()

    def scale(slot, half):
        def chunk(cc, _):
            sl = pl.ds(cc * 16, 16)
            co = ewv[slot, sl]
            for l in range(16):
                r = cc * 16 + l
                cb = jnp.full((16,), co[l], jnp.float32)
                for j in range(8):
                    sl2 = pl.ds(j * 16, 16)
                    rows[half, r, sl2] = rows[half, r, sl2] * cb
            return 0
        lax.fori_loop(0, 8, chunk, 0)

    nbase, nrem = NWIN // 32, NWIN % 32
    nw = nbase + (wid < nrem).astype(jnp.int32)

    issue_idx(0, wid)

    def body(i, _):
        par = lax.rem(i, 2)
        slot = lax.rem(i, 3)

        @pl.when(i >= 2)
        def _():
            wait_scatter(lax.rem(i + 1, 3), par)

        @pl.when(i + 1 < nw)
        def _():
            issue_idx(lax.rem(i + 1, 3), wid + (i + 1) * 32)

        @pl.when(i < nw)
        def _():
            wait_idx(slot)
            start_gather(slot, par)

        @pl.when(i >= 1)
        def _():
            pslot = lax.rem(i + 2, 3)
            ppar = 1 - par
            wait_gather(pslot, ppar)
            scale(pslot, ppar)
            start_scatter(pslot, ppar)
        return 0
    lax.fori_loop(0, nw + 1, body, 0)
    wait_scatter(lax.rem(nw + 2, 3), lax.rem(nw + 1, 2))

    plsc.subcore_barrier()

    for t in range(NP // 16 // EW):
        off = (s * 5 + t) * EW
        pltpu.sync_copy(agg_sh.at[pl.ds(off, EW)], rows.at[0])
        pltpu.sync_copy(rows.at[0], out_hbm.at[c, pl.ds(off, EW)])


def _agg_call(y, src, dst, ewf):
    mesh = plsc.VectorSubcoreMesh(core_axis_name="c", subcore_axis_name="s")
    f = functools.partial(
        pl.kernel,
        out_type=jax.ShapeDtypeStruct((2, NP, D), jnp.float32),
        mesh=mesh,
        compiler_params=pltpu.CompilerParams(needs_layout_passes=False),
        scratch_types=[
            pltpu.VMEM((3, EW), jnp.int32),
            pltpu.VMEM((3, EW), jnp.int32),
            pltpu.VMEM((3, EW), jnp.float32),
            pltpu.VMEM((2, EW, D), jnp.float32),
            pltpu.VMEM_SHARED((NP, D), jnp.float32),
            pltpu.SemaphoreType.DMA((3,)),
            pltpu.SemaphoreType.DMA((2,)),
            pltpu.SemaphoreType.DMA((2,)),
        ],
    )(_agg_body)
    return f(y, src, dst, ewf)


def _dec_body(aggp_ref, invd_ref, benc_ref, wd_ref, bdec_ref, out_ref):
    z = ((aggp_ref[0] + aggp_ref[1]) * invd_ref[...][:, None]
         + benc_ref[...])
    h = z * jax.nn.sigmoid(z)
    out_ref[...] = (jnp.dot(h, wd_ref[...], preferred_element_type=jnp.float32)
                    + bdec_ref[...])


def _dec_call(aggp, inv_d, b_enc, W_dec, b_dec):
    return pl.pallas_call(
        _dec_body,
        out_shape=jax.ShapeDtypeStruct((NP, D), jnp.float32),
    )(aggp, inv_d, b_enc, W_dec, b_dec)


def kernel(x, edge_index, edge_attr, W_edge, W_enc, b_enc, W_dec, b_dec):
    src = edge_index[0].astype(jnp.int32)
    dst = edge_index[1].astype(jnp.int32)
    ea_T = edge_attr.T.reshape(4, NWIN, EW)

    degp = _deg_call(src, dst)
    y, ew, inv2 = _enc_call(x, W_enc, ea_T, W_edge, degp)
    aggp = _agg_call(y, src, dst, ew.reshape(-1))
    out = _dec_call(aggp, inv2[1], b_enc, W_dec, b_dec)
    return out[:NN]


# speedup vs baseline: 1.9296x; 1.9197x over previous
import functools

import jax
import jax.numpy as jnp
from jax import lax
from jax.experimental import pallas as pl
from jax.experimental.pallas import tpu as pltpu
from jax.experimental.pallas import tpu_sc as plsc

NN = 10000
NP = 10240
NE = 320000
D = 128
EW = 128
NWIN = NE // EW


def _deg_body(src_hbm, dst_hbm, degp_hbm, idx_v, ones_v, stage_v,
              degs_sh, degd_sh):
    c = lax.axis_index("c")
    s = lax.axis_index("s")
    wid = c * 16 + s
    for j in range(8):
        ones_v[pl.ds(j * 16, 16)] = jnp.ones((16,), jnp.float32)
    zed = NP // 16
    def zrow(r, _):
        stage_v[pl.ds(r * 16, 16)] = jnp.zeros((16,), jnp.float32)
        return 0
    lax.fori_loop(0, zed // 16, zrow, 0)
    pltpu.sync_copy(stage_v, degs_sh.at[pl.ds(s * zed, zed)])
    pltpu.sync_copy(stage_v, degd_sh.at[pl.ds(s * zed, zed)])
    plsc.subcore_barrier()

    nbase, nrem = NWIN // 32, NWIN % 32
    nw = nbase + (wid < nrem).astype(jnp.int32)

    def body(i, _):
        base = (wid + i * 32) * EW
        pltpu.sync_copy(src_hbm.at[pl.ds(base, EW)], idx_v)
        pltpu.sync_copy(ones_v, degs_sh.at[idx_v], add=True)
        pltpu.sync_copy(dst_hbm.at[pl.ds(base, EW)], idx_v)
        pltpu.sync_copy(ones_v, degd_sh.at[idx_v], add=True)
        return 0
    lax.fori_loop(0, nw, body, 0)
    plsc.subcore_barrier()

    pltpu.sync_copy(degs_sh.at[pl.ds(s * zed, zed)], stage_v)
    pltpu.sync_copy(stage_v, degp_hbm.at[c, 0, pl.ds(s * zed, zed)])
    pltpu.sync_copy(degd_sh.at[pl.ds(s * zed, zed)], stage_v)
    pltpu.sync_copy(stage_v, degp_hbm.at[c, 1, pl.ds(s * zed, zed)])


def _deg_call(src, dst):
    mesh = plsc.VectorSubcoreMesh(core_axis_name="c", subcore_axis_name="s")
    zed = NP // 16
    f = functools.partial(
        pl.kernel,
        out_type=jax.ShapeDtypeStruct((2, 2, NP), jnp.float32),
        mesh=mesh,
        compiler_params=pltpu.CompilerParams(needs_layout_passes=False),
        scratch_types=[
            pltpu.VMEM((EW,), jnp.int32),
            pltpu.VMEM((EW,), jnp.float32),
            pltpu.VMEM((zed,), jnp.float32),
            pltpu.VMEM_SHARED((NP,), jnp.float32),
            pltpu.VMEM_SHARED((NP,), jnp.float32),
        ],
    )(_deg_body)
    return f(src, dst)


def _enc_body(x_ref, we_ref, ea_ref, wedge_ref, degp_ref,
              y_ref, ew_ref, inv_ref):
    deg = degp_ref[...]
    degsum = deg[0] + deg[1]
    inv = jnp.where(degsum > 0,
                    lax.rsqrt(jnp.maximum(degsum, 1e-12)),
                    jnp.zeros_like(degsum))
    inv_ref[...] = inv
    xs = x_ref[...] * inv[0, :NN, None]
    y_ref[...] = jnp.dot(xs, we_ref[...],
                         preferred_element_type=jnp.float32)
    ea = ea_ref[...]
    z = (ea[0] * wedge_ref[0, 0] + ea[1] * wedge_ref[1, 0]
         + ea[2] * wedge_ref[2, 0] + ea[3] * wedge_ref[3, 0])
    ew_ref[...] = jax.nn.sigmoid(z)


def _enc_call(x, W_enc, ea_T, W_edge, degp):
    return pl.pallas_call(
        _enc_body,
        out_shape=[
            jax.ShapeDtypeStruct((NN, D), jnp.float32),
            jax.ShapeDtypeStruct((NWIN, EW), jnp.float32),
            jax.ShapeDtypeStruct((2, NP), jnp.float32),
        ],
        in_specs=[
            pl.BlockSpec(memory_space=pltpu.VMEM),
            pl.BlockSpec(memory_space=pltpu.VMEM),
            pl.BlockSpec(memory_space=pltpu.VMEM),
            pl.BlockSpec(memory_space=pltpu.SMEM),
            pl.BlockSpec(memory_space=pltpu.VMEM),
        ],
    )(x, W_enc, ea_T, W_edge, degp)


NA = 10000


def _agg_body(y_hbm, src_hbm, dst_hbm, ew_hbm, out_hbm,
              sidx0, sidx1, sidx2, didx0, didx1, didx2, ewv0, ewv1, ewv2,
              rows0, rows1, rows2, agg_sh,
              semi0, semi1, semi2, semg0, semg1, semg2, sems0, sems1, sems2):
    c = lax.axis_index("c")
    s = lax.axis_index("s")
    wid = c * 16 + s
    sidx = (sidx0, sidx1, sidx2)
    didx = (didx0, didx1, didx2)
    ewv = (ewv0, ewv1, ewv2)
    rows = (rows0, rows1, rows2)
    semi = (semi0, semi1, semi2)
    semg = (semg0, semg1, semg2)
    sems = (sems0, sems1, sems2)

    def zrow(r, _):
        for j in range(8):
            rows0[r, pl.ds(j * 16, 16)] = jnp.zeros((16,), jnp.float32)
        return 0
    lax.fori_loop(0, EW, zrow, 0)
    base_off = s * 624
    for t in range(4):
        pltpu.sync_copy(rows0, agg_sh.at[pl.ds(base_off + t * EW, EW)])

    @pl.when(s < 15)
    def _():
        pltpu.sync_copy(rows0.at[pl.ds(0, 112)],
                        agg_sh.at[pl.ds(base_off + 4 * EW, 112)])

    @pl.when(s == 15)
    def _():
        pltpu.sync_copy(rows0, agg_sh.at[pl.ds(base_off + 4 * EW, EW)])
    plsc.subcore_barrier()

    def issue_idx(b, w):
        base = w * EW
        pltpu.async_copy(src_hbm.at[pl.ds(base, EW)], sidx[b], semi[b])
        pltpu.async_copy(dst_hbm.at[pl.ds(base, EW)], didx[b], semi[b])
        pltpu.async_copy(ew_hbm.at[pl.ds(base, EW)], ewv[b], semi[b])

    def wait_idx(b):
        pltpu.make_async_copy(src_hbm.at[pl.ds(0, EW)], sidx[b], semi[b]).wait()
        pltpu.make_async_copy(dst_hbm.at[pl.ds(0, EW)], didx[b], semi[b]).wait()
        pltpu.make_async_copy(ew_hbm.at[pl.ds(0, EW)], ewv[b], semi[b]).wait()

    def wait_gather(b):
        pltpu.make_async_copy(y_hbm.at[pl.ds(0, EW)], rows[b], semg[b]).wait()

    def wait_scatter(b):
        pltpu.make_async_copy(rows[b], agg_sh.at[pl.ds(0, EW)],
                              sems[b]).wait()

    def scale(b):
        def chunk(cc, _):
            co = ewv[b][pl.ds(cc * 16, 16)]
            for l in range(16):
                r = cc * 16 + l
                cb = jnp.full((16,), co[l], jnp.float32)
                for j in range(8):
                    sl2 = pl.ds(j * 16, 16)
                    rows[b][r, sl2] = rows[b][r, sl2] * cb
            return 0
        lax.fori_loop(0, 8, chunk, 0)

    ntrip = (NWIN // 32) // 3
    for b in range(3):
        issue_idx(b, wid + b * 32)

    def body(p, _):
        for b in range(3):
            wait_idx(b)
            pltpu.async_copy(y_hbm.at[sidx[b]], rows[b], semg[b])
        for b in range(3):
            wait_gather(b)
            scale(b)
            pltpu.async_copy(rows[b], agg_sh.at[didx[b]], sems[b], add=True)
        for b in range(3):
            wait_scatter(b)

            @pl.when(p < ntrip - 1)
            def _():
                issue_idx(b, wid + ((p + 1) * 3 + b) * 32)
        return 0
    lax.fori_loop(0, ntrip, body, 0)

    nrem = NWIN % 32

    @pl.when(wid < nrem)
    def _():
        issue_idx(0, wid + ntrip * 3 * 32)
        wait_idx(0)
        pltpu.async_copy(y_hbm.at[sidx[0]], rows[0], semg[0])
        wait_gather(0)
        scale(0)
        pltpu.async_copy(rows[0], agg_sh.at[didx[0]], sems[0], add=True)
        wait_scatter(0)

    plsc.subcore_barrier()

    for t in range(4):
        off = base_off + t * EW
        pltpu.sync_copy(agg_sh.at[pl.ds(off, EW)], rows0)
        pltpu.sync_copy(rows0, out_hbm.at[c, pl.ds(off, EW)])
    off5 = base_off + 4 * EW

    @pl.when(s < 15)
    def _():
        pltpu.sync_copy(agg_sh.at[pl.ds(off5, 112)], rows0.at[pl.ds(0, 112)])
        pltpu.sync_copy(rows0.at[pl.ds(0, 112)],
                        out_hbm.at[c, pl.ds(off5, 112)])

    @pl.when(s == 15)
    def _():
        pltpu.sync_copy(agg_sh.at[pl.ds(off5, EW)], rows0)
        pltpu.sync_copy(rows0, out_hbm.at[c, pl.ds(off5, EW)])


def _agg_call(y, src, dst, ewf):
    mesh = plsc.VectorSubcoreMesh(core_axis_name="c", subcore_axis_name="s")
    f = functools.partial(
        pl.kernel,
        out_type=jax.ShapeDtypeStruct((2, NA, D), jnp.float32),
        mesh=mesh,
        compiler_params=pltpu.CompilerParams(needs_layout_passes=False),
        scratch_types=(
            [pltpu.VMEM((EW,), jnp.int32) for _ in range(6)]
            + [pltpu.VMEM((EW,), jnp.float32) for _ in range(3)]
            + [pltpu.VMEM((EW, D), jnp.float32) for _ in range(3)]
            + [pltpu.VMEM_SHARED((NA, D), jnp.float32)]
            + [pltpu.SemaphoreType.DMA for _ in range(9)]
        ),
    )(_agg_body)
    return f(y, src, dst, ewf)


def _dec_body(aggp_ref, invd_ref, benc_ref, wd_ref, bdec_ref, out_ref):
    z = ((aggp_ref[0] + aggp_ref[1]) * invd_ref[...][:, None]
         + benc_ref[...])
    h = z * jax.nn.sigmoid(z)
    out_ref[...] = (jnp.dot(h, wd_ref[...], preferred_element_type=jnp.float32)
                    + bdec_ref[...])


def _dec_call(aggp, inv_d, b_enc, W_dec, b_dec):
    return pl.pallas_call(
        _dec_body,
        out_shape=jax.ShapeDtypeStruct((NA, D), jnp.float32),
    )(aggp, inv_d, b_enc, W_dec, b_dec)


def kernel(x, edge_index, edge_attr, W_edge, W_enc, b_enc, W_dec, b_dec):
    src = edge_index[0].astype(jnp.int32)
    dst = edge_index[1].astype(jnp.int32)
    ea_T = edge_attr.T.reshape(4, NWIN, EW)

    degp = _deg_call(src, dst)
    y, ew, inv2 = _enc_call(x, W_enc, ea_T, W_edge, degp)
    aggp = _agg_call(y, src, dst, ew.reshape(-1))
    out = _dec_call(aggp, inv2[1, :NA], b_enc, W_dec, b_dec)
    return out


# speedup vs baseline: 2.4314x; 1.2601x over previous
import functools

import jax
import jax.numpy as jnp
from jax import lax
from jax.experimental import pallas as pl
from jax.experimental.pallas import tpu as pltpu
from jax.experimental.pallas import tpu_sc as plsc

NN = 10000
NP = 10240
NE = 320000
D = 128
EW = 128
NWIN = NE // EW


def _deg_body(src_hbm, dst_hbm, degp_hbm, sidx0, sidx1, didx0, didx1,
              ones_v, stage_v, degs_sh, degd_sh,
              semi0, semi1, sema0, sema1):
    c = lax.axis_index("c")
    s = lax.axis_index("s")
    wid = c * 16 + s
    sidx = (sidx0, sidx1)
    didx = (didx0, didx1)
    semi = (semi0, semi1)
    sema = (sema0, sema1)
    for j in range(8):
        ones_v[pl.ds(j * 16, 16)] = jnp.ones((16,), jnp.float32)
    zed = NP // 16
    def zrow(r, _):
        stage_v[pl.ds(r * 16, 16)] = jnp.zeros((16,), jnp.float32)
        return 0
    lax.fori_loop(0, zed // 16, zrow, 0)
    pltpu.sync_copy(stage_v, degs_sh.at[pl.ds(s * zed, zed)])
    pltpu.sync_copy(stage_v, degd_sh.at[pl.ds(s * zed, zed)])
    plsc.subcore_barrier()

    def issue_idx(b, w):
        base = w * EW
        pltpu.async_copy(src_hbm.at[pl.ds(base, EW)], sidx[b], semi[b])
        pltpu.async_copy(dst_hbm.at[pl.ds(base, EW)], didx[b], semi[b])

    def wait_idx(b):
        pltpu.make_async_copy(src_hbm.at[pl.ds(0, EW)], sidx[b], semi[b]).wait()
        pltpu.make_async_copy(dst_hbm.at[pl.ds(0, EW)], didx[b], semi[b]).wait()

    def wait_adds(b):
        pltpu.make_async_copy(ones_v, degs_sh.at[pl.ds(0, EW)],
                              sema[b]).wait()
        pltpu.make_async_copy(ones_v, degd_sh.at[pl.ds(0, EW)],
                              sema[b]).wait()

    npair = (NWIN // 32) // 2
    issue_idx(0, wid)
    issue_idx(1, wid + 32)

    def body(p, _):
        for b in range(2):
            wait_idx(b)
            pltpu.async_copy(ones_v, degs_sh.at[sidx[b]], sema[b], add=True)
            pltpu.async_copy(ones_v, degd_sh.at[didx[b]], sema[b], add=True)
        for b in range(2):
            wait_adds(b)

            @pl.when(p < npair - 1)
            def _():
                issue_idx(b, wid + ((p + 1) * 2 + b) * 32)
        return 0
    lax.fori_loop(0, npair, body, 0)

    nrem = NWIN % 32

    @pl.when(wid < nrem)
    def _():
        issue_idx(0, wid + npair * 2 * 32)
        wait_idx(0)
        pltpu.async_copy(ones_v, degs_sh.at[sidx[0]], sema[0], add=True)
        pltpu.async_copy(ones_v, degd_sh.at[didx[0]], sema[0], add=True)
        wait_adds(0)

    plsc.subcore_barrier()

    pltpu.sync_copy(degs_sh.at[pl.ds(s * zed, zed)], stage_v)
    pltpu.sync_copy(stage_v, degp_hbm.at[c, 0, pl.ds(s * zed, zed)])
    pltpu.sync_copy(degd_sh.at[pl.ds(s * zed, zed)], stage_v)
    pltpu.sync_copy(stage_v, degp_hbm.at[c, 1, pl.ds(s * zed, zed)])


def _deg_call(src, dst):
    mesh = plsc.VectorSubcoreMesh(core_axis_name="c", subcore_axis_name="s")
    zed = NP // 16
    f = functools.partial(
        pl.kernel,
        out_type=jax.ShapeDtypeStruct((2, 2, NP), jnp.float32),
        mesh=mesh,
        compiler_params=pltpu.CompilerParams(needs_layout_passes=False),
        scratch_types=[
            pltpu.VMEM((EW,), jnp.int32),
            pltpu.VMEM((EW,), jnp.int32),
            pltpu.VMEM((EW,), jnp.int32),
            pltpu.VMEM((EW,), jnp.int32),
            pltpu.VMEM((EW,), jnp.float32),
            pltpu.VMEM((zed,), jnp.float32),
            pltpu.VMEM_SHARED((NP,), jnp.float32),
            pltpu.VMEM_SHARED((NP,), jnp.float32),
            pltpu.SemaphoreType.DMA,
            pltpu.SemaphoreType.DMA,
            pltpu.SemaphoreType.DMA,
            pltpu.SemaphoreType.DMA,
        ],
    )(_deg_body)
    return f(src, dst)


def _enc_body(x_ref, we_ref, ea_ref, wedge_ref, degp_ref,
              y_ref, ew_ref, inv_ref):
    deg = degp_ref[...]
    degsum = deg[0] + deg[1]
    inv = jnp.where(degsum > 0,
                    lax.rsqrt(jnp.maximum(degsum, 1e-12)),
                    jnp.zeros_like(degsum))
    inv_ref[...] = inv
    xs = x_ref[...] * inv[0, :NN, None]
    y_ref[...] = jnp.dot(xs, we_ref[...],
                         preferred_element_type=jnp.float32)
    ea = ea_ref[...]
    z = (ea[0] * wedge_ref[0, 0] + ea[1] * wedge_ref[1, 0]
         + ea[2] * wedge_ref[2, 0] + ea[3] * wedge_ref[3, 0])
    ew_ref[...] = jax.nn.sigmoid(z)


def _enc_call(x, W_enc, ea_T, W_edge, degp):
    return pl.pallas_call(
        _enc_body,
        out_shape=[
            jax.ShapeDtypeStruct((NN, D), jnp.float32),
            jax.ShapeDtypeStruct((NWIN, EW), jnp.float32),
            jax.ShapeDtypeStruct((2, NP), jnp.float32),
        ],
        in_specs=[
            pl.BlockSpec(memory_space=pltpu.VMEM),
            pl.BlockSpec(memory_space=pltpu.VMEM),
            pl.BlockSpec(memory_space=pltpu.VMEM),
            pl.BlockSpec(memory_space=pltpu.SMEM),
            pl.BlockSpec(memory_space=pltpu.VMEM),
        ],
    )(x, W_enc, ea_T, W_edge, degp)


NA = 10000


def _agg_body(y_hbm, src_hbm, dst_hbm, ew_hbm, out_hbm,
              sidx0, sidx1, sidx2, didx0, didx1, didx2, ewv0, ewv1, ewv2,
              rows0, rows1, rows2, agg_sh,
              semi0, semi1, semi2, semg0, semg1, semg2, sems0, sems1, sems2):
    c = lax.axis_index("c")
    s = lax.axis_index("s")
    wid = c * 16 + s
    sidx = (sidx0, sidx1, sidx2)
    didx = (didx0, didx1, didx2)
    ewv = (ewv0, ewv1, ewv2)
    rows = (rows0, rows1, rows2)
    semi = (semi0, semi1, semi2)
    semg = (semg0, semg1, semg2)
    sems = (sems0, sems1, sems2)

    def zrow(r, _):
        for j in range(8):
            rows0[r, pl.ds(j * 16, 16)] = jnp.zeros((16,), jnp.float32)
        return 0
    lax.fori_loop(0, EW, zrow, 0)
    base_off = s * 624
    for t in range(4):
        pltpu.sync_copy(rows0, agg_sh.at[pl.ds(base_off + t * EW, EW)])

    @pl.when(s < 15)
    def _():
        pltpu.sync_copy(rows0.at[pl.ds(0, 112)],
                        agg_sh.at[pl.ds(base_off + 4 * EW, 112)])

    @pl.when(s == 15)
    def _():
        pltpu.sync_copy(rows0, agg_sh.at[pl.ds(base_off + 4 * EW, EW)])
    plsc.subcore_barrier()

    def issue_idx(b, w):
        base = w * EW
        pltpu.async_copy(src_hbm.at[pl.ds(base, EW)], sidx[b], semi[b])
        pltpu.async_copy(dst_hbm.at[pl.ds(base, EW)], didx[b], semi[b])
        pltpu.async_copy(ew_hbm.at[pl.ds(base, EW)], ewv[b], semi[b])

    def wait_idx(b):
        pltpu.make_async_copy(src_hbm.at[pl.ds(0, EW)], sidx[b], semi[b]).wait()
        pltpu.make_async_copy(dst_hbm.at[pl.ds(0, EW)], didx[b], semi[b]).wait()
        pltpu.make_async_copy(ew_hbm.at[pl.ds(0, EW)], ewv[b], semi[b]).wait()

    def wait_gather(b):
        pltpu.make_async_copy(y_hbm.at[pl.ds(0, EW)], rows[b], semg[b]).wait()

    def wait_scatter(b):
        pltpu.make_async_copy(rows[b], agg_sh.at[pl.ds(0, EW)],
                              sems[b]).wait()

    def scale(b):
        def chunk(cc, _):
            co = ewv[b][pl.ds(cc * 16, 16)]
            for l in range(16):
                r = cc * 16 + l
                cb = jnp.full((16,), co[l], jnp.float32)
                for j in range(8):
                    sl2 = pl.ds(j * 16, 16)
                    rows[b][r, sl2] = rows[b][r, sl2] * cb
            return 0
        lax.fori_loop(0, 8, chunk, 0)

    ntrip = (NWIN // 32) // 3
    for b in range(3):
        issue_idx(b, wid + b * 32)

    def body(p, _):
        for b in range(3):
            wait_idx(b)
            pltpu.async_copy(y_hbm.at[sidx[b]], rows[b], semg[b])
        for b in range(3):
            wait_gather(b)
            scale(b)
            pltpu.async_copy(rows[b], agg_sh.at[didx[b]], sems[b], add=True)
        for b in range(3):
            wait_scatter(b)

            @pl.when(p < ntrip - 1)
            def _():
                issue_idx(b, wid + ((p + 1) * 3 + b) * 32)
        return 0
    lax.fori_loop(0, ntrip, body, 0)

    nrem = NWIN % 32

    @pl.when(wid < nrem)
    def _():
        issue_idx(0, wid + ntrip * 3 * 32)
        wait_idx(0)
        pltpu.async_copy(y_hbm.at[sidx[0]], rows[0], semg[0])
        wait_gather(0)
        scale(0)
        pltpu.async_copy(rows[0], agg_sh.at[didx[0]], sems[0], add=True)
        wait_scatter(0)

    plsc.subcore_barrier()

    for t in range(4):
        off = base_off + t * EW
        pltpu.sync_copy(agg_sh.at[pl.ds(off, EW)], rows0)
        pltpu.sync_copy(rows0, out_hbm.at[c, pl.ds(off, EW)])
    off5 = base_off + 4 * EW

    @pl.when(s < 15)
    def _():
        pltpu.sync_copy(agg_sh.at[pl.ds(off5, 112)], rows0.at[pl.ds(0, 112)])
        pltpu.sync_copy(rows0.at[pl.ds(0, 112)],
                        out_hbm.at[c, pl.ds(off5, 112)])

    @pl.when(s == 15)
    def _():
        pltpu.sync_copy(agg_sh.at[pl.ds(off5, EW)], rows0)
        pltpu.sync_copy(rows0, out_hbm.at[c, pl.ds(off5, EW)])


def _agg_call(y, src, dst, ewf):
    mesh = plsc.VectorSubcoreMesh(core_axis_name="c", subcore_axis_name="s")
    f = functools.partial(
        pl.kernel,
        out_type=jax.ShapeDtypeStruct((2, NA, D), jnp.float32),
        mesh=mesh,
        compiler_params=pltpu.CompilerParams(needs_layout_passes=False),
        scratch_types=(
            [pltpu.VMEM((EW,), jnp.int32) for _ in range(6)]
            + [pltpu.VMEM((EW,), jnp.float32) for _ in range(3)]
            + [pltpu.VMEM((EW, D), jnp.float32) for _ in range(3)]
            + [pltpu.VMEM_SHARED((NA, D), jnp.float32)]
            + [pltpu.SemaphoreType.DMA for _ in range(9)]
        ),
    )(_agg_body)
    return f(y, src, dst, ewf)


def _dec_body(aggp_ref, invd_ref, benc_ref, wd_ref, bdec_ref, out_ref):
    z = ((aggp_ref[0] + aggp_ref[1]) * invd_ref[...][:, None]
         + benc_ref[...])
    h = z * jax.nn.sigmoid(z)
    out_ref[...] = (jnp.dot(h, wd_ref[...], preferred_element_type=jnp.float32)
                    + bdec_ref[...])


def _dec_call(aggp, inv_d, b_enc, W_dec, b_dec):
    return pl.pallas_call(
        _dec_body,
        out_shape=jax.ShapeDtypeStruct((NA, D), jnp.float32),
    )(aggp, inv_d, b_enc, W_dec, b_dec)


def kernel(x, edge_index, edge_attr, W_edge, W_enc, b_enc, W_dec, b_dec):
    src = edge_index[0].astype(jnp.int32)
    dst = edge_index[1].astype(jnp.int32)
    ea_T = edge_attr.T.reshape(4, NWIN, EW)

    degp = _deg_call(src, dst)
    y, ew, inv2 = _enc_call(x, W_enc, ea_T, W_edge, degp)
    aggp = _agg_call(y, src, dst, ew.reshape(-1))
    out = _dec_call(aggp, inv2[1, :NA], b_enc, W_dec, b_dec)
    return out


# speedup vs baseline: 2.4437x; 1.0051x over previous
import functools

import jax
import jax.numpy as jnp
from jax import lax
from jax.experimental import pallas as pl
from jax.experimental.pallas import tpu as pltpu
from jax.experimental.pallas import tpu_sc as plsc

NN = 10000
NP = 10240
NE = 320000
D = 128
EW = 128
NWIN = NE // EW


def _deg_body(src_hbm, dst_hbm, degp_hbm, sidx0, sidx1, didx0, didx1,
              ones_v, stage_v, degs_sh, degd_sh,
              semi0, semi1, sema0, sema1):
    c = lax.axis_index("c")
    s = lax.axis_index("s")
    wid = c * 16 + s
    sidx = (sidx0, sidx1)
    didx = (didx0, didx1)
    semi = (semi0, semi1)
    sema = (sema0, sema1)
    for j in range(8):
        ones_v[pl.ds(j * 16, 16)] = jnp.ones((16,), jnp.float32)
    zed = NP // 16
    def zrow(r, _):
        stage_v[pl.ds(r * 16, 16)] = jnp.zeros((16,), jnp.float32)
        return 0
    lax.fori_loop(0, zed // 16, zrow, 0)
    pltpu.sync_copy(stage_v, degs_sh.at[pl.ds(s * zed, zed)])
    pltpu.sync_copy(stage_v, degd_sh.at[pl.ds(s * zed, zed)])
    plsc.subcore_barrier()

    def issue_idx(b, w):
        base = w * EW
        pltpu.async_copy(src_hbm.at[pl.ds(base, EW)], sidx[b], semi[b])
        pltpu.async_copy(dst_hbm.at[pl.ds(base, EW)], didx[b], semi[b])

    def wait_idx(b):
        pltpu.make_async_copy(src_hbm.at[pl.ds(0, EW)], sidx[b], semi[b]).wait()
        pltpu.make_async_copy(dst_hbm.at[pl.ds(0, EW)], didx[b], semi[b]).wait()

    def wait_adds(b):
        pltpu.make_async_copy(ones_v, degs_sh.at[pl.ds(0, EW)],
                              sema[b]).wait()
        pltpu.make_async_copy(ones_v, degd_sh.at[pl.ds(0, EW)],
                              sema[b]).wait()

    npair = (NWIN // 32) // 2
    issue_idx(0, wid)
    issue_idx(1, wid + 32)

    def body(p, _):
        for b in range(2):
            wait_idx(b)
            pltpu.async_copy(ones_v, degs_sh.at[sidx[b]], sema[b], add=True)
            pltpu.async_copy(ones_v, degd_sh.at[didx[b]], sema[b], add=True)
        for b in range(2):
            wait_adds(b)

            @pl.when(p < npair - 1)
            def _():
                issue_idx(b, wid + ((p + 1) * 2 + b) * 32)
        return 0
    lax.fori_loop(0, npair, body, 0)

    nrem = NWIN % 32

    @pl.when(wid < nrem)
    def _():
        issue_idx(0, wid + npair * 2 * 32)
        wait_idx(0)
        pltpu.async_copy(ones_v, degs_sh.at[sidx[0]], sema[0], add=True)
        pltpu.async_copy(ones_v, degd_sh.at[didx[0]], sema[0], add=True)
        wait_adds(0)

    plsc.subcore_barrier()

    pltpu.sync_copy(degs_sh.at[pl.ds(s * zed, zed)], stage_v)
    pltpu.sync_copy(stage_v, degp_hbm.at[c, 0, pl.ds(s * zed, zed)])
    pltpu.sync_copy(degd_sh.at[pl.ds(s * zed, zed)], stage_v)
    pltpu.sync_copy(stage_v, degp_hbm.at[c, 1, pl.ds(s * zed, zed)])


def _deg_call(src, dst):
    mesh = plsc.VectorSubcoreMesh(core_axis_name="c", subcore_axis_name="s")
    zed = NP // 16
    f = functools.partial(
        pl.kernel,
        out_type=jax.ShapeDtypeStruct((2, 2, NP), jnp.float32),
        mesh=mesh,
        compiler_params=pltpu.CompilerParams(needs_layout_passes=False),
        scratch_types=[
            pltpu.VMEM((EW,), jnp.int32),
            pltpu.VMEM((EW,), jnp.int32),
            pltpu.VMEM((EW,), jnp.int32),
            pltpu.VMEM((EW,), jnp.int32),
            pltpu.VMEM((EW,), jnp.float32),
            pltpu.VMEM((zed,), jnp.float32),
            pltpu.VMEM_SHARED((NP,), jnp.float32),
            pltpu.VMEM_SHARED((NP,), jnp.float32),
            pltpu.SemaphoreType.DMA,
            pltpu.SemaphoreType.DMA,
            pltpu.SemaphoreType.DMA,
            pltpu.SemaphoreType.DMA,
        ],
    )(_deg_body)
    return f(src, dst)


def _ew_body(ea_ref, wedge_ref, ew_ref):
    ea = ea_ref[...]
    z = (ea[0] * wedge_ref[0, 0] + ea[1] * wedge_ref[1, 0]
         + ea[2] * wedge_ref[2, 0] + ea[3] * wedge_ref[3, 0])
    ew_ref[...] = jax.nn.sigmoid(z)


def _ew_call(ea_T, W_edge):
    return pl.pallas_call(
        _ew_body,
        out_shape=jax.ShapeDtypeStruct((NWIN, EW), jnp.float32),
        in_specs=[
            pl.BlockSpec(memory_space=pltpu.VMEM),
            pl.BlockSpec(memory_space=pltpu.SMEM),
        ],
    )(ea_T, W_edge)


def _enc_body(x_ref, we_ref, degp_ref, y_ref, inv_ref):
    deg = degp_ref[...]
    degsum = deg[0] + deg[1]
    inv = jnp.where(degsum > 0,
                    lax.rsqrt(jnp.maximum(degsum, 1e-12)),
                    jnp.zeros_like(degsum))
    inv_ref[...] = inv
    xs = x_ref[...] * inv[0, :NN, None]
    y_ref[...] = jnp.dot(xs, we_ref[...],
                         preferred_element_type=jnp.float32)


def _enc_call(x, W_enc, degp):
    return pl.pallas_call(
        _enc_body,
        out_shape=[
            jax.ShapeDtypeStruct((NN, D), jnp.float32),
            jax.ShapeDtypeStruct((2, NP), jnp.float32),
        ],
    )(x, W_enc, degp)


NA = 10000


def _agg_body(y_hbm, src_hbm, dst_hbm, ew_hbm, out_hbm,
              sidx0, sidx1, sidx2, didx0, didx1, didx2, ewv0, ewv1, ewv2,
              rows0, rows1, rows2, agg_sh,
              semi0, semi1, semi2, semg0, semg1, semg2, sems0, sems1, sems2):
    c = lax.axis_index("c")
    s = lax.axis_index("s")
    wid = c * 16 + s
    sidx = (sidx0, sidx1, sidx2)
    didx = (didx0, didx1, didx2)
    ewv = (ewv0, ewv1, ewv2)
    rows = (rows0, rows1, rows2)
    semi = (semi0, semi1, semi2)
    semg = (semg0, semg1, semg2)
    sems = (sems0, sems1, sems2)

    def zrow(r, _):
        for j in range(8):
            rows0[r, pl.ds(j * 16, 16)] = jnp.zeros((16,), jnp.float32)
        return 0
    lax.fori_loop(0, EW, zrow, 0)
    base_off = s * 624
    for t in range(4):
        pltpu.sync_copy(rows0, agg_sh.at[pl.ds(base_off + t * EW, EW)])

    @pl.when(s < 15)
    def _():
        pltpu.sync_copy(rows0.at[pl.ds(0, 112)],
                        agg_sh.at[pl.ds(base_off + 4 * EW, 112)])

    @pl.when(s == 15)
    def _():
        pltpu.sync_copy(rows0, agg_sh.at[pl.ds(base_off + 4 * EW, EW)])
    plsc.subcore_barrier()

    def issue_idx(b, w):
        base = w * EW
        pltpu.async_copy(src_hbm.at[pl.ds(base, EW)], sidx[b], semi[b])
        pltpu.async_copy(dst_hbm.at[pl.ds(base, EW)], didx[b], semi[b])
        pltpu.async_copy(ew_hbm.at[pl.ds(base, EW)], ewv[b], semi[b])

    def wait_idx(b):
        pltpu.make_async_copy(src_hbm.at[pl.ds(0, EW)], sidx[b], semi[b]).wait()
        pltpu.make_async_copy(dst_hbm.at[pl.ds(0, EW)], didx[b], semi[b]).wait()
        pltpu.make_async_copy(ew_hbm.at[pl.ds(0, EW)], ewv[b], semi[b]).wait()

    def wait_gather(b):
        pltpu.make_async_copy(y_hbm.at[pl.ds(0, EW)], rows[b], semg[b]).wait()

    def wait_scatter(b):
        pltpu.make_async_copy(rows[b], agg_sh.at[pl.ds(0, EW)],
                              sems[b]).wait()

    def scale(b):
        def chunk(cc, _):
            co = ewv[b][pl.ds(cc * 16, 16)]
            for l in range(16):
                r = cc * 16 + l
                cb = jnp.full((16,), co[l], jnp.float32)
                for j in range(8):
                    sl2 = pl.ds(j * 16, 16)
                    rows[b][r, sl2] = rows[b][r, sl2] * cb
            return 0
        lax.fori_loop(0, 8, chunk, 0)

    ntrip = (NWIN // 32) // 3
    for b in range(3):
        issue_idx(b, wid + b * 32)

    def body(p, _):
        for b in range(3):
            wait_idx(b)
            pltpu.async_copy(y_hbm.at[sidx[b]], rows[b], semg[b])
        for b in range(3):
            wait_gather(b)
            scale(b)
            pltpu.async_copy(rows[b], agg_sh.at[didx[b]], sems[b], add=True)
        for b in range(3):
            wait_scatter(b)

            @pl.when(p < ntrip - 1)
            def _():
                issue_idx(b, wid + ((p + 1) * 3 + b) * 32)
        return 0
    lax.fori_loop(0, ntrip, body, 0)

    nrem = NWIN % 32

    @pl.when(wid < nrem)
    def _():
        issue_idx(0, wid + ntrip * 3 * 32)
        wait_idx(0)
        pltpu.async_copy(y_hbm.at[sidx[0]], rows[0], semg[0])
        wait_gather(0)
        scale(0)
        pltpu.async_copy(rows[0], agg_sh.at[didx[0]], sems[0], add=True)
        wait_scatter(0)

    plsc.subcore_barrier()

    for t in range(4):
        off = base_off + t * EW
        pltpu.sync_copy(agg_sh.at[pl.ds(off, EW)], rows0)
        pltpu.sync_copy(rows0, out_hbm.at[c, pl.ds(off, EW)])
    off5 = base_off + 4 * EW

    @pl.when(s < 15)
    def _():
        pltpu.sync_copy(agg_sh.at[pl.ds(off5, 112)], rows0.at[pl.ds(0, 112)])
        pltpu.sync_copy(rows0.at[pl.ds(0, 112)],
                        out_hbm.at[c, pl.ds(off5, 112)])

    @pl.when(s == 15)
    def _():
        pltpu.sync_copy(agg_sh.at[pl.ds(off5, EW)], rows0)
        pltpu.sync_copy(rows0, out_hbm.at[c, pl.ds(off5, EW)])


def _agg_call(y, src, dst, ewf):
    mesh = plsc.VectorSubcoreMesh(core_axis_name="c", subcore_axis_name="s")
    f = functools.partial(
        pl.kernel,
        out_type=jax.ShapeDtypeStruct((2, NA, D), jnp.float32),
        mesh=mesh,
        compiler_params=pltpu.CompilerParams(needs_layout_passes=False),
        scratch_types=(
            [pltpu.VMEM((EW,), jnp.int32) for _ in range(6)]
            + [pltpu.VMEM((EW,), jnp.float32) for _ in range(3)]
            + [pltpu.VMEM((EW, D), jnp.float32) for _ in range(3)]
            + [pltpu.VMEM_SHARED((NA, D), jnp.float32)]
            + [pltpu.SemaphoreType.DMA for _ in range(9)]
        ),
    )(_agg_body)
    return f(y, src, dst, ewf)


def _dec_body(aggp_ref, invd_ref, benc_ref, wd_ref, bdec_ref, out_ref):
    z = ((aggp_ref[0] + aggp_ref[1]) * invd_ref[...][:, None]
         + benc_ref[...])
    h = z * jax.nn.sigmoid(z)
    out_ref[...] = (jnp.dot(h, wd_ref[...], preferred_element_type=jnp.float32)
                    + bdec_ref[...])


def _dec_call(aggp, inv_d, b_enc, W_dec, b_dec):
    return pl.pallas_call(
        _dec_body,
        out_shape=jax.ShapeDtypeStruct((NA, D), jnp.float32),
    )(aggp, inv_d, b_enc, W_dec, b_dec)


def kernel(x, edge_index, edge_attr, W_edge, W_enc, b_enc, W_dec, b_dec):
    src = edge_index[0].astype(jnp.int32)
    dst = edge_index[1].astype(jnp.int32)
    ea_T = edge_attr.T.reshape(4, NWIN, EW)

    ew = _ew_call(ea_T, W_edge)
    degp = _deg_call(src, dst)
    y, inv2 = _enc_call(x, W_enc, degp)
    aggp = _agg_call(y, src, dst, ew.reshape(-1))
    out = _dec_call(aggp, inv2[1, :NA], b_enc, W_dec, b_dec)
    return out
